# baseline (speedup 1.0000x reference)
import jax
import jax.numpy as jnp
from jax import lax
from jax.experimental import pallas as pl
from jax.experimental.pallas import tpu as pltpu
from jax.experimental.pallas import tpu_sc as plsc

_H, _W = 128, 128
_HW = _H * _W
_L = 16


def _sc_scatter(idx_hbm, x_hbm, out_hbm, cell_v, row_v, col_v, x_v,
                buf_a, buf_b, buf_c, sem_a, sem_b, sem_c, sem_x):
    B, M, N = x_hbm.shape
    nhalf = N // 2
    nc = 2
    wid = lax.axis_index("s") * nc + lax.axis_index("c")
    b = wid // 2
    nlo = (wid % 2) * nhalf

    pltpu.sync_copy(idx_hbm.at[b], cell_v)
    pltpu.async_copy(x_hbm.at[b], x_v, sem_x)

    def rc_body(mi, _):
        cell = cell_v[pl.ds(mi * _L, _L)]
        row_v[pl.ds(mi * _L, _L)] = lax.shift_right_logical(cell, 7)
        col_v[pl.ds(mi * _L, _L)] = cell & 127
        return 0

    lax.fori_loop(0, M // _L, rc_body, 0)

    def zero_body(r, _):
        for c in range(_W // _L):
            buf_a[r, pl.ds(c * _L, _L)] = jnp.zeros((_L,), jnp.float32)
            buf_b[r, pl.ds(c * _L, _L)] = jnp.zeros((_L,), jnp.float32)
            buf_c[r, pl.ds(c * _L, _L)] = jnp.zeros((_L,), jnp.float32)
        return 0

    lax.fori_loop(0, _H, zero_body, 0)

    pltpu.make_async_copy(x_hbm.at[b], x_v, sem_x).wait()

    lane = lax.iota(jnp.int32, _L)

    def scatter_plane(n, buf):
        ncol = jnp.full((_L,), n, jnp.int32)

        def m_body(mo, _):
            for u in range(4):
                mi = mo * 4 + u
                sl = pl.ds(mi * _L, _L)
                val = plsc.load_gather(x_v, [mi * _L + lane, ncol])
                plsc.store_scatter(buf, [row_v[sl], col_v[sl]], val)
            return 0

        lax.fori_loop(0, M // (_L * 4), m_body, 0)

    bufs = (buf_a, buf_b, buf_c)
    sems = (sem_a, sem_b, sem_c)
    for r in range(nhalf):
        buf = bufs[r % 3]
        sem = sems[r % 3]
        if r >= 3:
            pltpu.make_async_copy(
                buf, out_hbm.at[b, nlo + r - 3], sem).wait()
        scatter_plane(nlo + r, buf)
        pltpu.async_copy(buf, out_hbm.at[b, nlo + r], sem)

    for r in range(nhalf - 3, nhalf):
        pltpu.make_async_copy(
            bufs[r % 3], out_hbm.at[b, nlo + r], sems[r % 3]).wait()


def kernel(x, spatial_size, location):
    B, M, N = x.shape
    H, W = _H, _W
    HW = H * W
    index = (location[:, :, 0] * spatial_size[1] + location[:, :, 1]) % HW
    index = index.astype(jnp.int32)

    mesh = plsc.VectorSubcoreMesh(core_axis_name="c", subcore_axis_name="s")
    scatter = pl.kernel(
        _sc_scatter,
        mesh=mesh,
        out_type=jax.ShapeDtypeStruct((B, N, H, W), jnp.float32),
        scratch_types=[
            pltpu.VMEM((M,), jnp.int32),
            pltpu.VMEM((M,), jnp.int32),
            pltpu.VMEM((M,), jnp.int32),
            pltpu.VMEM((M, N), jnp.float32),
            pltpu.VMEM((H, W), jnp.float32),
            pltpu.VMEM((H, W), jnp.float32),
            pltpu.VMEM((H, W), jnp.float32),
            pltpu.SemaphoreType.DMA,
            pltpu.SemaphoreType.DMA,
            pltpu.SemaphoreType.DMA,
            pltpu.SemaphoreType.DMA,
        ],
        compiler_params=pltpu.CompilerParams(needs_layout_passes=False),
    )
    return scatter(index, x)


# speedup vs baseline: 1.0932x; 1.0932x over previous
import jax
import jax.numpy as jnp
from jax import lax
from jax.experimental import pallas as pl
from jax.experimental.pallas import tpu as pltpu
from jax.experimental.pallas import tpu_sc as plsc

_H, _W = 128, 128
_HW = _H * _W
_L = 16


def _sc_scatter(idx_hbm, x_hbm, out_hbm, cell_v, row_v, col_v, x_v,
                buf_a, buf_b, sem_a, sem_b, sem_x):
    B, M, N = x_hbm.shape
    nhalf = N // 2
    nc = 2
    wid = lax.axis_index("s") * nc + lax.axis_index("c")
    b = wid // 2
    nlo = (wid % 2) * nhalf

    pltpu.sync_copy(idx_hbm.at[b], cell_v)
    pltpu.async_copy(x_hbm.at[b], x_v, sem_x)

    def rc_body(mi, _):
        cell = cell_v[pl.ds(mi * _L, _L)]
        row_v[pl.ds(mi * _L, _L)] = lax.shift_right_logical(cell, 7)
        col_v[pl.ds(mi * _L, _L)] = cell & 127
        return 0

    lax.fori_loop(0, M // _L, rc_body, 0)

    def zero_body(r, _):
        for c in range(_W // _L):
            buf_a[r, pl.ds(c * _L, _L)] = jnp.zeros((_L,), jnp.float32)
            buf_b[r, pl.ds(c * _L, _L)] = jnp.zeros((_L,), jnp.float32)
        return 0

    lax.fori_loop(0, _H, zero_body, 0)

    pltpu.make_async_copy(x_hbm.at[b], x_v, sem_x).wait()

    lane = lax.iota(jnp.int32, _L)

    def scatter_plane(n, buf):
        ncol = jnp.full((_L,), n, jnp.int32)

        def m_body(mo, _):
            for u in range(4):
                mi = mo * 4 + u
                sl = pl.ds(mi * _L, _L)
                val = plsc.load_gather(x_v, [mi * _L + lane, ncol])
                plsc.store_scatter(buf, [row_v[sl], col_v[sl]], val)
            return 0

        lax.fori_loop(0, M // (_L * 4), m_body, 0)

    def plane_pair(r2, _):
        n0 = nlo + 2 * r2
        n1 = nlo + 2 * r2 + 1

        @pl.when(r2 > 0)
        def _():
            pltpu.make_async_copy(buf_a, out_hbm.at[b, n0 - 2], sem_a).wait()

        scatter_plane(n0, buf_a)
        pltpu.async_copy(buf_a, out_hbm.at[b, n0], sem_a)

        @pl.when(r2 > 0)
        def _():
            pltpu.make_async_copy(buf_b, out_hbm.at[b, n1 - 2], sem_b).wait()

        scatter_plane(n1, buf_b)
        pltpu.async_copy(buf_b, out_hbm.at[b, n1], sem_b)
        return 0

    lax.fori_loop(0, nhalf // 2, plane_pair, 0)

    pltpu.make_async_copy(
        buf_a, out_hbm.at[b, nlo + nhalf - 2], sem_a).wait()
    pltpu.make_async_copy(
        buf_b, out_hbm.at[b, nlo + nhalf - 1], sem_b).wait()


def kernel(x, spatial_size, location):
    B, M, N = x.shape
    H, W = _H, _W
    HW = H * W
    index = (location[:, :, 0] * spatial_size[1] + location[:, :, 1]) % HW
    index = index.astype(jnp.int32)

    mesh = plsc.VectorSubcoreMesh(core_axis_name="c", subcore_axis_name="s")
    scatter = pl.kernel(
        _sc_scatter,
        mesh=mesh,
        out_type=jax.ShapeDtypeStruct((B, N, H, W), jnp.float32),
        scratch_types=[
            pltpu.VMEM((M,), jnp.int32),
            pltpu.VMEM((M,), jnp.int32),
            pltpu.VMEM((M,), jnp.int32),
            pltpu.VMEM((M, N), jnp.float32),
            pltpu.VMEM((H, W), jnp.float32),
            pltpu.VMEM((H, W), jnp.float32),
            pltpu.SemaphoreType.DMA,
            pltpu.SemaphoreType.DMA,
            pltpu.SemaphoreType.DMA,
        ],
        compiler_params=pltpu.CompilerParams(needs_layout_passes=False),
    )
    return scatter(index, x)
